# baseline (speedup 1.0000x reference)
import functools

import jax
import jax.numpy as jnp
from jax import lax
from jax.experimental import pallas as pl
from jax.experimental.pallas import tpu as pltpu
from jax.experimental.pallas import tpu_sc as plsc

NUM_GRAPHS = 256
HIDDEN = 512
DIM_TARGET = 32
N_NODES = 100000

NC = 2
NS = 16

COLS = HIDDEN // NC
CCH = COLS // 16
ROWS_PER_TILE = 6240
BLK = 96
NBLK = ROWS_PER_TILE // BLK
TAIL_START = NS * ROWS_PER_TILE
TAIL_TILES = (N_NODES - TAIL_START) // 16

_mesh = plsc.VectorSubcoreMesh(core_axis_name="c", subcore_axis_name="s")


@functools.partial(
    pl.kernel,
    mesh=_mesh,
    out_type=jax.ShapeDtypeStruct((NC, NS, NUM_GRAPHS, COLS), jnp.float32),
    scratch_types=[
        pltpu.VMEM((2, BLK, COLS), jnp.float32),
        pltpu.VMEM((2, BLK + 16), jnp.int32),
        pltpu.VMEM((NUM_GRAPHS + 1, COLS), jnp.float32),
        pltpu.SemaphoreType.DMA,
    ],
)
def _segsum_sc(x_hbm, batch_hbm, out_hbm, rows_v, idx_v, acc, sem):
    c = lax.axis_index("c")
    s = lax.axis_index("s")
    cbase = c * COLS
    rbase = s * ROWS_PER_TILE

    zero16 = jnp.zeros((16,), jnp.float32)

    def zero_body(r, carry):
        for k in range(CCH):
            acc[r, pl.ds(k * 16, 16)] = zero16
        return carry

    lax.fori_loop(0, NUM_GRAPHS + 1, zero_body, 0)

    def process_group(b, j):
        v = idx_v[b, pl.ds(j, 16)]
        v0 = v[0]
        v15 = v[15]
        uniform = v0 == v15
        tgt = jnp.where(uniform, v0, NUM_GRAPHS)

        for k in range(CCH):
            sl = pl.ds(k * 16, 16)
            vals = [rows_v[b, j + r, sl] for r in range(16)]
            while len(vals) > 1:
                nxt = [
                    vals[i] + vals[i + 1] for i in range(0, len(vals) - 1, 2)
                ]
                if len(vals) % 2:
                    nxt.append(vals[-1])
                vals = nxt
            acc[tgt, sl] = acc[tgt, sl] + vals[0]

        @pl.when(~uniform)
        def _():
            def row_body(r, carry2):
                sr = idx_v[b, pl.ds(j + r, 16)][0]
                for k in range(CCH):
                    sl = pl.ds(k * 16, 16)
                    acc[sr, sl] = acc[sr, sl] + rows_v[b, j + r, sl]
                return carry2

            lax.fori_loop(0, 16, row_body, 0)

    def start_block(i, b):
        off = rbase + i * BLK
        pltpu.async_copy(
            x_hbm.at[pl.ds(off, BLK), pl.ds(cbase, COLS)], rows_v.at[b], sem
        )
        pltpu.async_copy(
            batch_hbm.at[pl.ds(off, BLK)], idx_v.at[b, pl.ds(0, BLK)], sem
        )

    def wait_block(i, b):
        off = rbase + i * BLK
        pltpu.make_async_copy(
            x_hbm.at[pl.ds(off, BLK), pl.ds(cbase, COLS)], rows_v.at[b], sem
        ).wait()
        pltpu.make_async_copy(
            batch_hbm.at[pl.ds(off, BLK)], idx_v.at[b, pl.ds(0, BLK)], sem
        ).wait()

    start_block(0, 0)

    def block_body(i, carry):
        b = i & 1
        wait_block(i, b)

        @pl.when(i + 1 < NBLK)
        def _():
            start_block(i + 1, 1 - b)

        for gidx in range(BLK // 16):
            process_group(b, gidx * 16)
        return carry

    lax.fori_loop(0, NBLK, block_body, 0)

    @pl.when(s < TAIL_TILES)
    def _tail():
        off = TAIL_START + s * 16
        pltpu.sync_copy(
            x_hbm.at[pl.ds(off, 16), pl.ds(cbase, COLS)],
            rows_v.at[0, pl.ds(0, 16)],
        )
        pltpu.sync_copy(batch_hbm.at[pl.ds(off, 16)], idx_v.at[0, pl.ds(0, 16)])

        def row_body(r, carry2):
            sr = idx_v[0, pl.ds(r, 16)][0]
            for k in range(CCH):
                sl = pl.ds(k * 16, 16)
                acc[sr, sl] = acc[sr, sl] + rows_v[0, r, sl]
            return carry2

        lax.fori_loop(0, 16, row_body, 0)

    pltpu.sync_copy(acc.at[pl.ds(0, NUM_GRAPHS)], out_hbm.at[c, s])


def _reduce_body(p_ref, pooled_ref):
    pooled_ref[...] = jnp.sum(p_ref[0], axis=0)


_reduce_tc = pl.pallas_call(
    _reduce_body,
    grid=(NC,),
    in_specs=[
        pl.BlockSpec((1, NS, NUM_GRAPHS, COLS), lambda i: (i, 0, 0, 0))
    ],
    out_specs=pl.BlockSpec((NUM_GRAPHS, COLS), lambda i: (0, i)),
    out_shape=jax.ShapeDtypeStruct((NUM_GRAPHS, HIDDEN), jnp.float32),
)


def _tail_body(p_ref, w_ref, b_ref, mix_ref):
    logits = lax.dot_general(
        p_ref[...], w_ref[...],
        dimension_numbers=(((1,), (1,)), ((), ())),
        preferred_element_type=jnp.float32,
    ) + b_ref[...]
    m = jnp.max(logits, axis=-1, keepdims=True)
    e = jnp.exp(logits - m)
    mix = e / jnp.sum(e, axis=-1, keepdims=True)
    mix_ref[...] = jnp.clip(mix, 1e-8, 1.0)


_tail_tc = pl.pallas_call(
    _tail_body,
    out_shape=jax.ShapeDtypeStruct((NUM_GRAPHS, DIM_TARGET), jnp.float32),
)


def kernel(x, batch, W, b):
    batch = batch.astype(jnp.int32)
    part = _segsum_sc(x, batch)
    pooled = _reduce_tc(part)
    mix = _tail_tc(pooled, W, b.reshape(1, DIM_TARGET))
    return mix, pooled


# speedup vs baseline: 1.3445x; 1.3445x over previous
import functools

import jax
import jax.numpy as jnp
from jax import lax
from jax.experimental import pallas as pl
from jax.experimental.pallas import tpu as pltpu
from jax.experimental.pallas import tpu_sc as plsc

NUM_GRAPHS = 256
HIDDEN = 512
DIM_TARGET = 32
N_NODES = 100000

NC = 2
NS = 16

COLS = HIDDEN // NC
CCH = COLS // 16
ROWS_PER_TILE = 6240
BLK = 96
NBLK = ROWS_PER_TILE // BLK
TAIL_START = NS * ROWS_PER_TILE
TAIL_TILES = (N_NODES - TAIL_START) // 16

_mesh = plsc.VectorSubcoreMesh(core_axis_name="c", subcore_axis_name="s")


@functools.partial(
    pl.kernel,
    mesh=_mesh,
    out_type=jax.ShapeDtypeStruct((NC, NS, NUM_GRAPHS, COLS), jnp.float32),
    scratch_types=[
        pltpu.VMEM((2, BLK, COLS), jnp.float32),
        pltpu.VMEM((2, BLK + 16), jnp.int32),
        pltpu.VMEM((NUM_GRAPHS + 1, COLS), jnp.float32),
        pltpu.SemaphoreType.DMA,
    ],
)
def _segsum_sc(x_hbm, batch_hbm, out_hbm, rows_v, idx_v, acc, sem):
    c = lax.axis_index("c")
    s = lax.axis_index("s")
    cbase = c * COLS
    rbase = s * ROWS_PER_TILE

    zero16 = jnp.zeros((16,), jnp.float32)

    def zero_body(r, carry):
        for k in range(CCH):
            acc[r, pl.ds(k * 16, 16)] = zero16
        return carry

    lax.fori_loop(0, NUM_GRAPHS + 1, zero_body, 0)

    def process_group(b, j):
        v = idx_v[b, pl.ds(j, 16)]
        v0 = v[0]
        v15 = v[15]
        uniform = v0 == v15
        tgt = jnp.where(uniform, v0, NUM_GRAPHS)

        for q in range(CCH // 8):
            a = [None] * 8
            for t in range(8):
                a[t] = rows_v[b, j, pl.ds((q * 8 + t) * 16, 16)]
            for r in range(1, 16):
                for t in range(8):
                    a[t] = a[t] + rows_v[b, j + r, pl.ds((q * 8 + t) * 16, 16)]
            for t in range(8):
                sl = pl.ds((q * 8 + t) * 16, 16)
                acc[tgt, sl] = acc[tgt, sl] + a[t]

        @pl.when(~uniform)
        def _():
            def row_body(r, carry2):
                sr = idx_v[b, pl.ds(j + r, 16)][0]
                for k in range(CCH):
                    sl = pl.ds(k * 16, 16)
                    acc[sr, sl] = acc[sr, sl] + rows_v[b, j + r, sl]
                return carry2

            lax.fori_loop(0, 16, row_body, 0)

    def start_block(i, b):
        off = rbase + i * BLK
        pltpu.async_copy(
            x_hbm.at[pl.ds(off, BLK), pl.ds(cbase, COLS)], rows_v.at[b], sem
        )
        pltpu.async_copy(
            batch_hbm.at[pl.ds(off, BLK)], idx_v.at[b, pl.ds(0, BLK)], sem
        )

    def wait_block(i, b):
        off = rbase + i * BLK
        pltpu.make_async_copy(
            x_hbm.at[pl.ds(off, BLK), pl.ds(cbase, COLS)], rows_v.at[b], sem
        ).wait()
        pltpu.make_async_copy(
            batch_hbm.at[pl.ds(off, BLK)], idx_v.at[b, pl.ds(0, BLK)], sem
        ).wait()

    start_block(0, 0)

    def block_body(i, carry):
        b = i & 1
        wait_block(i, b)

        @pl.when(i + 1 < NBLK)
        def _():
            start_block(i + 1, 1 - b)

        for gidx in range(BLK // 16):
            process_group(b, gidx * 16)
        return carry

    lax.fori_loop(0, NBLK, block_body, 0)

    @pl.when(s < TAIL_TILES)
    def _tail():
        off = TAIL_START + s * 16
        pltpu.sync_copy(
            x_hbm.at[pl.ds(off, 16), pl.ds(cbase, COLS)],
            rows_v.at[0, pl.ds(0, 16)],
        )
        pltpu.sync_copy(batch_hbm.at[pl.ds(off, 16)], idx_v.at[0, pl.ds(0, 16)])

        def row_body(r, carry2):
            sr = idx_v[0, pl.ds(r, 16)][0]
            for k in range(CCH):
                sl = pl.ds(k * 16, 16)
                acc[sr, sl] = acc[sr, sl] + rows_v[0, r, sl]
            return carry2

        lax.fori_loop(0, 16, row_body, 0)

    pltpu.sync_copy(acc.at[pl.ds(0, NUM_GRAPHS)], out_hbm.at[c, s])


def _reduce_body(p_ref, pooled_ref):
    pooled_ref[...] = jnp.sum(p_ref[0], axis=0)


_reduce_tc = pl.pallas_call(
    _reduce_body,
    grid=(NC,),
    in_specs=[
        pl.BlockSpec((1, NS, NUM_GRAPHS, COLS), lambda i: (i, 0, 0, 0))
    ],
    out_specs=pl.BlockSpec((NUM_GRAPHS, COLS), lambda i: (0, i)),
    out_shape=jax.ShapeDtypeStruct((NUM_GRAPHS, HIDDEN), jnp.float32),
)


def _tail_body(p_ref, w_ref, b_ref, mix_ref):
    logits = lax.dot_general(
        p_ref[...], w_ref[...],
        dimension_numbers=(((1,), (1,)), ((), ())),
        preferred_element_type=jnp.float32,
    ) + b_ref[...]
    m = jnp.max(logits, axis=-1, keepdims=True)
    e = jnp.exp(logits - m)
    mix = e / jnp.sum(e, axis=-1, keepdims=True)
    mix_ref[...] = jnp.clip(mix, 1e-8, 1.0)


_tail_tc = pl.pallas_call(
    _tail_body,
    out_shape=jax.ShapeDtypeStruct((NUM_GRAPHS, DIM_TARGET), jnp.float32),
)


def kernel(x, batch, W, b):
    batch = batch.astype(jnp.int32)
    part = _segsum_sc(x, batch)
    pooled = _reduce_tc(part)
    mix = _tail_tc(pooled, W, b.reshape(1, DIM_TARGET))
    return mix, pooled


# speedup vs baseline: 1.3711x; 1.0198x over previous
import functools

import jax
import jax.numpy as jnp
from jax import lax
from jax.experimental import pallas as pl
from jax.experimental.pallas import tpu as pltpu
from jax.experimental.pallas import tpu_sc as plsc

NUM_GRAPHS = 256
HIDDEN = 512
DIM_TARGET = 32
N_NODES = 100000

NC = 2
NS = 16

COLS = HIDDEN // NC
CCH = COLS // 16
ROWS_PER_TILE = 6240
BLK = 96
NBLK = ROWS_PER_TILE // BLK
TAIL_START = NS * ROWS_PER_TILE
TAIL_TILES = (N_NODES - TAIL_START) // 16

_mesh = plsc.VectorSubcoreMesh(core_axis_name="c", subcore_axis_name="s")


@functools.partial(
    pl.kernel,
    mesh=_mesh,
    out_type=jax.ShapeDtypeStruct((NC, NS, NUM_GRAPHS, COLS), jnp.float32),
    scratch_types=[
        pltpu.VMEM((2, BLK, COLS), jnp.float32),
        pltpu.VMEM((2, BLK + 16), jnp.int32),
        pltpu.VMEM((NUM_GRAPHS + 1, COLS), jnp.float32),
        pltpu.SemaphoreType.DMA,
    ],
)
def _segsum_sc(x_hbm, batch_hbm, out_hbm, rows_v, idx_v, acc, sem):
    c = lax.axis_index("c")
    s = lax.axis_index("s")
    cbase = c * COLS
    rbase = s * ROWS_PER_TILE

    zero16 = jnp.zeros((16,), jnp.float32)

    def zero_body(r, carry):
        for k in range(CCH):
            acc[r, pl.ds(k * 16, 16)] = zero16
        return carry

    lax.fori_loop(0, NUM_GRAPHS + 1, zero_body, 0)

    def process_group(b, j):
        v = idx_v[b, pl.ds(j, 16)]
        v0 = v[0]
        v15 = v[15]
        uniform = v0 == v15
        tgt = jnp.where(uniform, v0, NUM_GRAPHS)

        for q in range(CCH // 4):
            a = [None] * 4
            for t in range(4):
                a[t] = rows_v[b, j, pl.ds((q * 4 + t) * 16, 16)]
            for r in range(1, 16):
                for t in range(4):
                    a[t] = a[t] + rows_v[b, j + r, pl.ds((q * 4 + t) * 16, 16)]
            for t in range(4):
                sl = pl.ds((q * 4 + t) * 16, 16)
                acc[tgt, sl] = acc[tgt, sl] + a[t]

        @pl.when(~uniform)
        def _():
            def row_body(r, carry2):
                sr = idx_v[b, pl.ds(j + r, 16)][0]
                for k in range(CCH):
                    sl = pl.ds(k * 16, 16)
                    acc[sr, sl] = acc[sr, sl] + rows_v[b, j + r, sl]
                return carry2

            lax.fori_loop(0, 16, row_body, 0)

    def start_block(i, b):
        off = rbase + i * BLK
        pltpu.async_copy(
            x_hbm.at[pl.ds(off, BLK), pl.ds(cbase, COLS)], rows_v.at[b], sem
        )
        pltpu.async_copy(
            batch_hbm.at[pl.ds(off, BLK)], idx_v.at[b, pl.ds(0, BLK)], sem
        )

    def wait_block(i, b):
        off = rbase + i * BLK
        pltpu.make_async_copy(
            x_hbm.at[pl.ds(off, BLK), pl.ds(cbase, COLS)], rows_v.at[b], sem
        ).wait()
        pltpu.make_async_copy(
            batch_hbm.at[pl.ds(off, BLK)], idx_v.at[b, pl.ds(0, BLK)], sem
        ).wait()

    start_block(0, 0)

    def block_body(i, carry):
        b = i & 1
        wait_block(i, b)

        @pl.when(i + 1 < NBLK)
        def _():
            start_block(i + 1, 1 - b)

        for gidx in range(BLK // 16):
            process_group(b, gidx * 16)
        return carry

    lax.fori_loop(0, NBLK, block_body, 0)

    @pl.when(s < TAIL_TILES)
    def _tail():
        off = TAIL_START + s * 16
        pltpu.sync_copy(
            x_hbm.at[pl.ds(off, 16), pl.ds(cbase, COLS)],
            rows_v.at[0, pl.ds(0, 16)],
        )
        pltpu.sync_copy(batch_hbm.at[pl.ds(off, 16)], idx_v.at[0, pl.ds(0, 16)])

        def row_body(r, carry2):
            sr = idx_v[0, pl.ds(r, 16)][0]
            for k in range(CCH):
                sl = pl.ds(k * 16, 16)
                acc[sr, sl] = acc[sr, sl] + rows_v[0, r, sl]
            return carry2

        lax.fori_loop(0, 16, row_body, 0)

    pltpu.sync_copy(acc.at[pl.ds(0, NUM_GRAPHS)], out_hbm.at[c, s])


def _reduce_body(p_ref, pooled_ref):
    pooled_ref[...] = jnp.sum(p_ref[0], axis=0)


_reduce_tc = pl.pallas_call(
    _reduce_body,
    grid=(NC,),
    in_specs=[
        pl.BlockSpec((1, NS, NUM_GRAPHS, COLS), lambda i: (i, 0, 0, 0))
    ],
    out_specs=pl.BlockSpec((NUM_GRAPHS, COLS), lambda i: (0, i)),
    out_shape=jax.ShapeDtypeStruct((NUM_GRAPHS, HIDDEN), jnp.float32),
)


def _tail_body(p_ref, w_ref, b_ref, mix_ref):
    logits = lax.dot_general(
        p_ref[...], w_ref[...],
        dimension_numbers=(((1,), (1,)), ((), ())),
        preferred_element_type=jnp.float32,
    ) + b_ref[...]
    m = jnp.max(logits, axis=-1, keepdims=True)
    e = jnp.exp(logits - m)
    mix = e / jnp.sum(e, axis=-1, keepdims=True)
    mix_ref[...] = jnp.clip(mix, 1e-8, 1.0)


_tail_tc = pl.pallas_call(
    _tail_body,
    out_shape=jax.ShapeDtypeStruct((NUM_GRAPHS, DIM_TARGET), jnp.float32),
)


def kernel(x, batch, W, b):
    batch = batch.astype(jnp.int32)
    part = _segsum_sc(x, batch)
    pooled = _reduce_tc(part)
    mix = _tail_tc(pooled, W, b.reshape(1, DIM_TARGET))
    return mix, pooled
